# baseline (speedup 1.0000x reference)
import functools

import jax
import jax.numpy as jnp
import numpy as np
from jax import lax
from jax.experimental import pallas as pl
from jax.experimental.pallas import tpu as pltpu
from jax.experimental.pallas import tpu_sc as plsc

Dim = 2
R = 1.0
sigma = float(np.sqrt(2.0))
gamma = 1.0
NSTEP = 50
total_time = 0.2
dt = total_time / NSTEP
E1 = float(np.exp(-gamma * dt).astype(np.float32))

LANES = 128
RB = 8
BLK = RB * LANES

NSAMP = 65536
NWORK = 32
PER_W = NSAMP // NWORK
SBLK = 512
NSBLK = PER_W // SBLK
SGRP = SBLK // 16


def _sc_phase1_body(x0_hbm, dw_hbm, ub_hbm,
                    o_xs0, o_xs1, o_df0, o_df1, o_epk, o_A, o_exf,
                    dwv, x0v, ubv, v_xs0, v_xs1, v_df0, v_df1, v_epk, v_A, v_exf):
    wid = lax.axis_index("s") * 2 + lax.axis_index("c")
    pltpu.sync_copy(ub_hbm, ubv)
    uv = ubv[...]

    for blk in range(NSBLK):
        base = wid * PER_W + blk * SBLK
        pltpu.sync_copy(dw_hbm.at[:, :, pl.ds(base, SBLK)], dwv)
        pltpu.sync_copy(x0_hbm.at[:, pl.ds(base, SBLK)], x0v)

        def group(g, carry):
            gl = pl.ds(g * 16, 16)
            x00 = x0v[0, gl]
            x01 = x0v[1, gl]
            zeros = jnp.zeros((16,), jnp.float32)
            ones = jnp.ones((16,), jnp.float32)

            def step(t, st):
                x0_, x1_, alivef, exitedf, A, ep, epk, xs0, xs1, sdf0, sdf1 = st
                d0 = sigma * dwv[0, t, gl]
                d1 = sigma * dwv[1, t, gl]
                g0 = uv * x0_ * dt
                g1 = uv * x1_ * dt
                t0 = x0_ + (g0 + d0)
                t1 = x1_ + (g1 + d1)
                exn = jnp.where(t0 * t0 + t1 * t1 < R * R, 1.0, 0.0)
                nx0 = (x0_ + g0) + d0
                nx1 = (x1_ + g1) + d1
                aliveN_f = alivef * exn
                newexit_f = alivef - aliveN_f
                nb = newexit_f > 0.0
                ab = aliveN_f > 0.0
                xs0 = jnp.where(nb, x0_, xs0)
                xs1 = jnp.where(nb, x1_, xs1)
                sdf0 = jnp.where(nb, d0, sdf0)
                sdf1 = jnp.where(nb, d1, sdf1)
                epk = jnp.where(nb, ep, epk)
                exitedf = exitedf + newexit_f
                x0_ = jnp.where(ab, nx0, x0_)
                x1_ = jnp.where(ab, nx1, x1_)
                w = (uv * uv + 2.0) * (x0_ * x0_ + x1_ * x1_) - 2.0 * Dim
                A = A + aliveN_f * (w * ep)
                ep = ep * E1
                return (x0_, x1_, aliveN_f, exitedf, A, ep, epk, xs0, xs1, sdf0, sdf1)

            st = (x00, x01, ones, zeros, zeros, ones,
                  zeros, zeros, zeros, zeros, zeros)
            st = lax.fori_loop(1, NSTEP, step, st)
            x0_, x1_, alivef, exitedf, A, ep, epk, xs0, xs1, sdf0, sdf1 = st
            eb = exitedf > 0.0
            xs0 = jnp.where(eb, xs0, x0_)
            xs1 = jnp.where(eb, xs1, x1_)
            epk = jnp.where(eb, epk, ep)
            sl = pl.ds(g * 16, 16)
            v_xs0[sl] = xs0
            v_xs1[sl] = xs1
            v_df0[sl] = sdf0
            v_df1[sl] = sdf1
            v_epk[sl] = epk
            v_A[sl] = A
            v_exf[sl] = exitedf
            return carry

        lax.fori_loop(0, SGRP, group, 0)
        pltpu.sync_copy(v_xs0, o_xs0.at[pl.ds(base, SBLK)])
        pltpu.sync_copy(v_xs1, o_xs1.at[pl.ds(base, SBLK)])
        pltpu.sync_copy(v_df0, o_df0.at[pl.ds(base, SBLK)])
        pltpu.sync_copy(v_df1, o_df1.at[pl.ds(base, SBLK)])
        pltpu.sync_copy(v_epk, o_epk.at[pl.ds(base, SBLK)])
        pltpu.sync_copy(v_A, o_A.at[pl.ds(base, SBLK)])
        pltpu.sync_copy(v_exf, o_exf.at[pl.ds(base, SBLK)])


_SC_OUT = [jax.ShapeDtypeStruct((NSAMP,), jnp.float32)] * 7

_sc_phase1 = functools.partial(
    pl.kernel,
    mesh=plsc.VectorSubcoreMesh(core_axis_name="c", subcore_axis_name="s"),
    out_type=_SC_OUT,
    scratch_types=[
        pltpu.VMEM((2, NSTEP, SBLK), jnp.float32),
        pltpu.VMEM((2, SBLK), jnp.float32),
        pltpu.VMEM((16,), jnp.float32),
    ] + [pltpu.VMEM((SBLK,), jnp.float32)] * 7,
)(_sc_phase1_body)


def _rho_tile(xe0, xe1, dr0, dr1, df0, df1):
    a = dr0 * dr0 + dr1 * dr1
    b = 2.0 * (dr0 * df0 + dr1 * df1)
    c = (2.0 * dr0 * xe0 + df0 * df0) + (2.0 * dr1 * xe1 + df1 * df1)
    d = 2.0 * (df0 * xe0 + df1 * xe1)
    e = (xe0 * xe0 + xe1 * xe1) - R ** 2
    p = (8.0 * a * c - 3.0 * (b * b)) / (8.0 * (a * a))
    q = (b * b * b - 4.0 * a * b * c + 8.0 * (a * a) * d) / (8.0 * (a * a * a))
    sign_q = jnp.sign(q)
    D0 = c * c - 3.0 * b * d + 12.0 * a * e
    D1 = (2.0 * (c * c * c) - 9.0 * b * c * d + 27.0 * (b * b) * e
          + 27.0 * a * (d * d) - 72.0 * a * c * e)
    D2 = D1 * D1 - 4.0 * (D0 * D0 * D0)
    signal_D2 = jnp.ceil((jnp.sign(D2) + 1.0) / 2.0)
    QQ = (D1 + jnp.sqrt(jnp.abs(D2))) / 2.0
    Q = jnp.sign(QQ) * jnp.abs(QQ) ** (1.0 / 3.0)
    S_plus = 0.5 * jnp.sqrt(jnp.abs((Q + D0 / Q) / (3.0 * a) - 2.0 * p / 3.0))
    m = jnp.minimum(jnp.sqrt(jnp.abs(D1 * D1 / 4.0 / (D0 * D0 * D0))), 1.0)
    phi = jax.lax.atan2(jnp.sqrt((1.0 - m) * (m + 1.0)), m)
    S_minus = 0.5 * jnp.sqrt(jnp.abs(2.0 * jnp.sqrt(jnp.abs(D0)) * jnp.cos(phi / 3.0) / (3.0 * a)
                                     - 2.0 * p / 3.0))
    S = signal_D2 * S_plus + (1.0 - signal_D2) * S_minus
    temp = -4.0 * (S * S) - 2.0 * p + jnp.abs(q / S)
    sqrt_rho = 0.5 * jnp.sqrt(jnp.abs(temp)) - b / (4.0 * a) - sign_q * S
    bad = (1.0 - sqrt_rho) * sqrt_rho < 0.0
    new_temp = -4.0 * (S * S) - 2.0 * p - jnp.abs(q / S)
    new_sqrt_rho = 0.5 * jnp.sqrt(jnp.abs(new_temp)) - b / (4.0 * a) + sign_q * S
    sqrt_rho_final = jnp.where(bad, new_sqrt_rho, sqrt_rho)
    return sqrt_rho_final * sqrt_rho_final, jnp.abs(sqrt_rho_final)


def _tc_phase2_body(x0_ref, dw0_ref, u_ref,
                    xs0_ref, xs1_ref, df0_ref, df1_ref, epk_ref, A_ref, exf_ref,
                    y_ref):
    uf = u_ref[0, 0]
    x00 = x0_ref[0]
    x01 = x0_ref[1]
    d00 = sigma * dw0_ref[0]
    d01 = sigma * dw0_ref[1]
    g0 = uf * x00 * dt
    g1 = uf * x01 * dt
    t0 = x00 + (g0 + d00)
    t1 = x01 + (g1 + d01)
    exit0 = t0 * t0 + t1 * t1 >= R * R
    w0 = (uf * uf + 2.0) * (x00 * x00 + x01 * x01) - 2.0 * Dim

    xs0 = xs0_ref[...]
    xs1 = xs1_ref[...]
    sdf0 = df0_ref[...]
    sdf1 = df1_ref[...]
    epk = epk_ref[...]
    A = A_ref[...]
    exited = exf_ref[...] > 0.0

    xe0 = jnp.where(exited, xs0, 0.1)
    xe1 = jnp.where(exited, xs1, 0.1)
    dr0 = jnp.where(exited, uf * xs0 * dt, 0.01)
    dr1 = jnp.where(exited, uf * xs1 * dt, 0.01)
    df0 = jnp.where(exited, sdf0, 0.01)
    df1 = jnp.where(exited, sdf1, 0.01)
    rho_q, srho_q = _rho_tile(xe0, xe1, dr0, dr1, df0, df1)
    rho = jnp.where(exited, rho_q, 0.0)
    srho = jnp.where(exited, srho_q, 0.0)
    xk0 = xs0 + uf * xs0 * dt * rho + sdf0 * srho
    xk1 = xs1 + uf * xs1 * dt * rho + sdf1 * srho
    nrm = xk0 * xk0 + xk1 * xk1
    term = epk * (rho * ((uf * uf + 2.0) * nrm - 2.0 * Dim) * dt
                  + jnp.exp(-gamma * dt * rho) * nrm)

    xe0b = jnp.where(exit0, x00, 0.1)
    xe1b = jnp.where(exit0, x01, 0.1)
    dr0b = jnp.where(exit0, uf * x00 * dt, 0.01)
    dr1b = jnp.where(exit0, uf * x01 * dt, 0.01)
    df0b = jnp.where(exit0, d00, 0.01)
    df1b = jnp.where(exit0, d01, 0.01)
    rho0, _ = _rho_tile(xe0b, xe1b, dr0b, dr1b, df0b, df1b)
    coef0 = jnp.where(exit0, rho0, 1.0)
    D0x = jnp.exp(-gamma * dt * coef0)
    y_ref[...] = coef0 * w0 * dt + D0x * (dt * A + term)


@jax.jit
def kernel(x0, dw, u):
    nsamp = x0.shape[0]
    nblk = nsamp // BLK
    rows = nsamp // LANES
    ub = jnp.full((16,), u, jnp.float32)
    x0t = x0.transpose(1, 0)
    dwt = dw.transpose(1, 2, 0)
    ph1 = _sc_phase1(x0t, dwt, ub)
    xs0, xs1, df0, df1, epk, A, exf = [v.reshape(rows, LANES) for v in ph1]

    x0r = x0t.reshape(2, rows, LANES)
    dw0r = dwt[:, 0, :].reshape(2, rows, LANES)
    u2d = jnp.reshape(u.astype(jnp.float32), (1, 1))

    row_spec = pl.BlockSpec((RB, LANES), lambda i: (i, 0))
    pair_spec = pl.BlockSpec((2, RB, LANES), lambda i: (0, i, 0))
    y = pl.pallas_call(
        _tc_phase2_body,
        grid=(nblk,),
        in_specs=[pair_spec, pair_spec,
                  pl.BlockSpec(memory_space=pltpu.SMEM),
                  row_spec, row_spec, row_spec, row_spec, row_spec,
                  row_spec, row_spec],
        out_specs=row_spec,
        out_shape=jax.ShapeDtypeStruct((rows, LANES), jnp.float32),
    )(x0r, dw0r, u2d, xs0, xs1, df0, df1, epk, A, exf)
    return y.reshape(nsamp, 1)


# speedup vs baseline: 1.2737x; 1.2737x over previous
import functools

import jax
import jax.numpy as jnp
import numpy as np
from jax import lax
from jax.experimental import pallas as pl
from jax.experimental.pallas import tpu as pltpu
from jax.experimental.pallas import tpu_sc as plsc

Dim = 2
R = 1.0
sigma = float(np.sqrt(2.0))
gamma = 1.0
NSTEP = 50
total_time = 0.2
dt = total_time / NSTEP
E1 = float(np.exp(-gamma * dt).astype(np.float32))

LANES = 128
RB = 8
BLK = RB * LANES

NSAMP = 65536
NA = 32768
NB = NSAMP - NA
NWORK = 32
PER_W = NA // NWORK
SBLK = 512
NSBLK = PER_W // SBLK
SGRP = SBLK // 16


def _sc_phase1_body(x0_hbm, dw_hbm, ub_hbm,
                    o_xs0, o_xs1, o_df0, o_df1, o_epk, o_A, o_exf,
                    dwv, x0v, ubv, v_xs0, v_xs1, v_df0, v_df1, v_epk, v_A, v_exf):
    wid = lax.axis_index("s") * 2 + lax.axis_index("c")
    pltpu.sync_copy(ub_hbm, ubv)
    uv = ubv[...]

    for blk in range(NSBLK):
        base = wid * PER_W + blk * SBLK
        pltpu.sync_copy(dw_hbm.at[:, :, pl.ds(base, SBLK)], dwv)
        pltpu.sync_copy(x0_hbm.at[:, pl.ds(base, SBLK)], x0v)

        def group(g, carry):
            gl = pl.ds(g * 16, 16)
            x00 = x0v[0, gl]
            x01 = x0v[1, gl]
            zeros = jnp.zeros((16,), jnp.float32)
            ones = jnp.ones((16,), jnp.float32)

            def step(t, st):
                x0_, x1_, alivef, exitedf, A, ep, epk, xs0, xs1, sdf0, sdf1 = st
                d0 = sigma * dwv[0, t, gl]
                d1 = sigma * dwv[1, t, gl]
                g0 = uv * x0_ * dt
                g1 = uv * x1_ * dt
                t0 = x0_ + (g0 + d0)
                t1 = x1_ + (g1 + d1)
                exn = jnp.where(t0 * t0 + t1 * t1 < R * R, 1.0, 0.0)
                nx0 = (x0_ + g0) + d0
                nx1 = (x1_ + g1) + d1
                aliveN_f = alivef * exn
                newexit_f = alivef - aliveN_f
                nb = newexit_f > 0.0
                ab = aliveN_f > 0.0
                xs0 = jnp.where(nb, x0_, xs0)
                xs1 = jnp.where(nb, x1_, xs1)
                sdf0 = jnp.where(nb, d0, sdf0)
                sdf1 = jnp.where(nb, d1, sdf1)
                epk = jnp.where(nb, ep, epk)
                exitedf = exitedf + newexit_f
                x0_ = jnp.where(ab, nx0, x0_)
                x1_ = jnp.where(ab, nx1, x1_)
                w = (uv * uv + 2.0) * (x0_ * x0_ + x1_ * x1_) - 2.0 * Dim
                A = A + aliveN_f * (w * ep)
                ep = ep * E1
                return (x0_, x1_, aliveN_f, exitedf, A, ep, epk, xs0, xs1, sdf0, sdf1)

            st = (x00, x01, ones, zeros, zeros, ones,
                  zeros, zeros, zeros, zeros, zeros)
            st = lax.fori_loop(1, NSTEP, step, st)
            x0_, x1_, alivef, exitedf, A, ep, epk, xs0, xs1, sdf0, sdf1 = st
            eb = exitedf > 0.0
            xs0 = jnp.where(eb, xs0, x0_)
            xs1 = jnp.where(eb, xs1, x1_)
            epk = jnp.where(eb, epk, ep)
            v_xs0[gl] = xs0
            v_xs1[gl] = xs1
            v_df0[gl] = sdf0
            v_df1[gl] = sdf1
            v_epk[gl] = epk
            v_A[gl] = A
            v_exf[gl] = exitedf
            return carry

        lax.fori_loop(0, SGRP, group, 0)
        pltpu.sync_copy(v_xs0, o_xs0.at[pl.ds(base, SBLK)])
        pltpu.sync_copy(v_xs1, o_xs1.at[pl.ds(base, SBLK)])
        pltpu.sync_copy(v_df0, o_df0.at[pl.ds(base, SBLK)])
        pltpu.sync_copy(v_df1, o_df1.at[pl.ds(base, SBLK)])
        pltpu.sync_copy(v_epk, o_epk.at[pl.ds(base, SBLK)])
        pltpu.sync_copy(v_A, o_A.at[pl.ds(base, SBLK)])
        pltpu.sync_copy(v_exf, o_exf.at[pl.ds(base, SBLK)])


_SC_OUT = [jax.ShapeDtypeStruct((NA,), jnp.float32)] * 7

_sc_phase1 = functools.partial(
    pl.kernel,
    mesh=plsc.VectorSubcoreMesh(core_axis_name="c", subcore_axis_name="s"),
    out_type=_SC_OUT,
    scratch_types=[
        pltpu.VMEM((2, NSTEP, SBLK), jnp.float32),
        pltpu.VMEM((2, SBLK), jnp.float32),
        pltpu.VMEM((16,), jnp.float32),
    ] + [pltpu.VMEM((SBLK,), jnp.float32)] * 7,
)(_sc_phase1_body)


def _rho_tile(xe0, xe1, dr0, dr1, df0, df1):
    a = dr0 * dr0 + dr1 * dr1
    b = 2.0 * (dr0 * df0 + dr1 * df1)
    c = (2.0 * dr0 * xe0 + df0 * df0) + (2.0 * dr1 * xe1 + df1 * df1)
    d = 2.0 * (df0 * xe0 + df1 * xe1)
    e = (xe0 * xe0 + xe1 * xe1) - R ** 2
    p = (8.0 * a * c - 3.0 * (b * b)) / (8.0 * (a * a))
    q = (b * b * b - 4.0 * a * b * c + 8.0 * (a * a) * d) / (8.0 * (a * a * a))
    sign_q = jnp.sign(q)
    D0 = c * c - 3.0 * b * d + 12.0 * a * e
    D1 = (2.0 * (c * c * c) - 9.0 * b * c * d + 27.0 * (b * b) * e
          + 27.0 * a * (d * d) - 72.0 * a * c * e)
    D2 = D1 * D1 - 4.0 * (D0 * D0 * D0)
    signal_D2 = jnp.ceil((jnp.sign(D2) + 1.0) / 2.0)
    QQ = (D1 + jnp.sqrt(jnp.abs(D2))) / 2.0
    Q = jnp.sign(QQ) * jnp.abs(QQ) ** (1.0 / 3.0)
    S_plus = 0.5 * jnp.sqrt(jnp.abs((Q + D0 / Q) / (3.0 * a) - 2.0 * p / 3.0))
    m = jnp.minimum(jnp.sqrt(jnp.abs(D1 * D1 / 4.0 / (D0 * D0 * D0))), 1.0)
    phi = jax.lax.atan2(jnp.sqrt((1.0 - m) * (m + 1.0)), m)
    S_minus = 0.5 * jnp.sqrt(jnp.abs(2.0 * jnp.sqrt(jnp.abs(D0)) * jnp.cos(phi / 3.0) / (3.0 * a)
                                     - 2.0 * p / 3.0))
    S = signal_D2 * S_plus + (1.0 - signal_D2) * S_minus
    temp = -4.0 * (S * S) - 2.0 * p + jnp.abs(q / S)
    sqrt_rho = 0.5 * jnp.sqrt(jnp.abs(temp)) - b / (4.0 * a) - sign_q * S
    bad = (1.0 - sqrt_rho) * sqrt_rho < 0.0
    new_temp = -4.0 * (S * S) - 2.0 * p - jnp.abs(q / S)
    new_sqrt_rho = 0.5 * jnp.sqrt(jnp.abs(new_temp)) - b / (4.0 * a) + sign_q * S
    sqrt_rho_final = jnp.where(bad, new_sqrt_rho, sqrt_rho)
    return sqrt_rho_final * sqrt_rho_final, jnp.abs(sqrt_rho_final)


def _phase2_tail(uf, x00, x01, d00, d01, exit0, w0,
                 xs0, xs1, sdf0, sdf1, epk, A, exited):
    xe0 = jnp.where(exited, xs0, 0.1)
    xe1 = jnp.where(exited, xs1, 0.1)
    dr0 = jnp.where(exited, uf * xs0 * dt, 0.01)
    dr1 = jnp.where(exited, uf * xs1 * dt, 0.01)
    df0 = jnp.where(exited, sdf0, 0.01)
    df1 = jnp.where(exited, sdf1, 0.01)
    rho_q, srho_q = _rho_tile(xe0, xe1, dr0, dr1, df0, df1)
    rho = jnp.where(exited, rho_q, 0.0)
    srho = jnp.where(exited, srho_q, 0.0)
    xk0 = xs0 + uf * xs0 * dt * rho + sdf0 * srho
    xk1 = xs1 + uf * xs1 * dt * rho + sdf1 * srho
    nrm = xk0 * xk0 + xk1 * xk1
    term = epk * (rho * ((uf * uf + 2.0) * nrm - 2.0 * Dim) * dt
                  + jnp.exp(-gamma * dt * rho) * nrm)

    xe0b = jnp.where(exit0, x00, 0.1)
    xe1b = jnp.where(exit0, x01, 0.1)
    dr0b = jnp.where(exit0, uf * x00 * dt, 0.01)
    dr1b = jnp.where(exit0, uf * x01 * dt, 0.01)
    df0b = jnp.where(exit0, d00, 0.01)
    df1b = jnp.where(exit0, d01, 0.01)
    rho0, _ = _rho_tile(xe0b, xe1b, dr0b, dr1b, df0b, df1b)
    coef0 = jnp.where(exit0, rho0, 1.0)
    D0x = jnp.exp(-gamma * dt * coef0)
    return coef0 * w0 * dt + D0x * (dt * A + term)


def _tc_full_body(x0_ref, dwt_ref, u_ref, y_ref):
    uf = u_ref[0, 0]
    x00 = x0_ref[0]
    x01 = x0_ref[1]

    d00 = sigma * dwt_ref[0, 0]
    d01 = sigma * dwt_ref[1, 0]
    g0 = uf * x00 * dt
    g1 = uf * x01 * dt
    t0 = x00 + (g0 + d00)
    t1 = x01 + (g1 + d01)
    exit0 = t0 * t0 + t1 * t1 >= R * R
    w0 = (uf * uf + 2.0) * (x00 * x00 + x01 * x01) - 2.0 * Dim

    zeros = jnp.zeros_like(x00)

    def step(t, st):
        x0_, x1_, alivef, exitedf, A, ep, epk, xs0, xs1, sdf0, sdf1 = st
        alive = alivef > 0.0
        d0 = sigma * dwt_ref[0, t]
        d1 = sigma * dwt_ref[1, t]
        g0 = uf * x0_ * dt
        g1 = uf * x1_ * dt
        t0 = x0_ + (g0 + d0)
        t1 = x1_ + (g1 + d1)
        ex = t0 * t0 + t1 * t1 >= R * R
        nx0 = (x0_ + g0) + d0
        nx1 = (x1_ + g1) + d1
        newexit = alive & ex
        xs0 = jnp.where(newexit, x0_, xs0)
        xs1 = jnp.where(newexit, x1_, xs1)
        sdf0 = jnp.where(newexit, d0, sdf0)
        sdf1 = jnp.where(newexit, d1, sdf1)
        epk = jnp.where(newexit, ep, epk)
        exitedf = jnp.where(newexit, 1.0, exitedf)
        aliveN = alive & jnp.logical_not(ex)
        x0_ = jnp.where(aliveN, nx0, x0_)
        x1_ = jnp.where(aliveN, nx1, x1_)
        w = (uf * uf + 2.0) * (x0_ * x0_ + x1_ * x1_) - 2.0 * Dim
        A = A + jnp.where(aliveN, w * ep, 0.0)
        ep = ep * E1
        alivef = jnp.where(aliveN, 1.0, 0.0)
        return (x0_, x1_, alivef, exitedf, A, ep, epk, xs0, xs1, sdf0, sdf1)

    st = (x00, x01, jnp.ones_like(x00), zeros,
          zeros, jnp.ones_like(x00), zeros, zeros, zeros, zeros, zeros)
    st = jax.lax.fori_loop(1, NSTEP, step, st, unroll=2)
    x0_, x1_, alivef, exitedf, A, ep, epk, xs0, xs1, sdf0, sdf1 = st
    exited = exitedf > 0.0
    xs0 = jnp.where(exited, xs0, x0_)
    xs1 = jnp.where(exited, xs1, x1_)
    epk = jnp.where(exited, epk, ep)

    y_ref[...] = _phase2_tail(uf, x00, x01, d00, d01, exit0, w0,
                              xs0, xs1, sdf0, sdf1, epk, A, exited)


def _tc_phase2_body(x0_ref, dw0_ref, u_ref,
                    xs0_ref, xs1_ref, df0_ref, df1_ref, epk_ref, A_ref, exf_ref,
                    y_ref):
    uf = u_ref[0, 0]
    x00 = x0_ref[0]
    x01 = x0_ref[1]
    d00 = sigma * dw0_ref[0]
    d01 = sigma * dw0_ref[1]
    g0 = uf * x00 * dt
    g1 = uf * x01 * dt
    t0 = x00 + (g0 + d00)
    t1 = x01 + (g1 + d01)
    exit0 = t0 * t0 + t1 * t1 >= R * R
    w0 = (uf * uf + 2.0) * (x00 * x00 + x01 * x01) - 2.0 * Dim

    y_ref[...] = _phase2_tail(uf, x00, x01, d00, d01, exit0, w0,
                              xs0_ref[...], xs1_ref[...], df0_ref[...],
                              df1_ref[...], epk_ref[...], A_ref[...],
                              exf_ref[...] > 0.0)


@jax.jit
def kernel(x0, dw, u):
    rows = NSAMP // LANES
    rows_a = NA // LANES
    blk_a = NA // BLK
    blk_b = NB // BLK

    x0t = x0.transpose(1, 0)
    dwt = dw.transpose(1, 2, 0)
    ub = jnp.full((16,), u, jnp.float32)
    u2d = jnp.reshape(u.astype(jnp.float32), (1, 1))

    x0r = x0t.reshape(2, rows, LANES)
    dwt4 = dwt.reshape(2, NSTEP, rows, LANES)
    dw0r = dwt4[:, 0]

    ph1 = _sc_phase1(x0t, dwt, ub)
    xs0, xs1, df0, df1, epk, A, exf = [v.reshape(rows_a, LANES) for v in ph1]

    pair_b = pl.BlockSpec((2, RB, LANES), lambda i: (0, blk_a + i, 0))
    y_b = pl.pallas_call(
        _tc_full_body,
        grid=(blk_b,),
        in_specs=[pair_b,
                  pl.BlockSpec((2, NSTEP, RB, LANES), lambda i: (0, 0, blk_a + i, 0)),
                  pl.BlockSpec(memory_space=pltpu.SMEM)],
        out_specs=pl.BlockSpec((RB, LANES), lambda i: (i, 0)),
        out_shape=jax.ShapeDtypeStruct((NB // LANES, LANES), jnp.float32),
    )(x0r, dwt4, u2d)

    row_spec = pl.BlockSpec((RB, LANES), lambda i: (i, 0))
    pair_a = pl.BlockSpec((2, RB, LANES), lambda i: (0, i, 0))
    y_a = pl.pallas_call(
        _tc_phase2_body,
        grid=(blk_a,),
        in_specs=[pair_a, pair_a,
                  pl.BlockSpec(memory_space=pltpu.SMEM),
                  row_spec, row_spec, row_spec, row_spec, row_spec,
                  row_spec, row_spec],
        out_specs=row_spec,
        out_shape=jax.ShapeDtypeStruct((rows_a, LANES), jnp.float32),
    )(x0r, dw0r, u2d, xs0, xs1, df0, df1, epk, A, exf)

    y = jnp.concatenate([y_a.reshape(-1), y_b.reshape(-1)])
    return y.reshape(NSAMP, 1)


# speedup vs baseline: 1.5051x; 1.1817x over previous
import functools

import jax
import jax.numpy as jnp
import numpy as np
from jax import lax
from jax.experimental import pallas as pl
from jax.experimental.pallas import tpu as pltpu
from jax.experimental.pallas import tpu_sc as plsc

Dim = 2
R = 1.0
sigma = float(np.sqrt(2.0))
gamma = 1.0
NSTEP = 50
total_time = 0.2
dt = total_time / NSTEP
E1 = float(np.exp(-gamma * dt).astype(np.float32))

LANES = 128
RB = 16
BLK = RB * LANES

NSAMP = 65536
NA = 32768
NB = NSAMP - NA
NWORK = 32
PER_W = NA // NWORK
SBLK = 512
NSBLK = PER_W // SBLK
SGRP = SBLK // 16


def _sc_phase1_body(x0_hbm, dw_hbm, ub_hbm,
                    o_xs0, o_xs1, o_df0, o_df1, o_epk, o_A, o_exf,
                    dwv, x0v, ubv, v_xs0, v_xs1, v_df0, v_df1, v_epk, v_A, v_exf):
    wid = lax.axis_index("s") * 2 + lax.axis_index("c")
    pltpu.sync_copy(ub_hbm, ubv)
    uv = ubv[...]

    for blk in range(NSBLK):
        base = wid * PER_W + blk * SBLK
        pltpu.sync_copy(dw_hbm.at[:, :, pl.ds(base, SBLK)], dwv)
        pltpu.sync_copy(x0_hbm.at[:, pl.ds(base, SBLK)], x0v)

        def group(g, carry):
            gl = pl.ds(g * 16, 16)
            x00 = x0v[0, gl]
            x01 = x0v[1, gl]
            zeros = jnp.zeros((16,), jnp.float32)
            ones = jnp.ones((16,), jnp.float32)

            def step(t, st):
                x0_, x1_, alivef, exitedf, A, ep, epk, xs0, xs1, sdf0, sdf1 = st
                d0 = sigma * dwv[0, t, gl]
                d1 = sigma * dwv[1, t, gl]
                g0 = uv * x0_ * dt
                g1 = uv * x1_ * dt
                t0 = x0_ + (g0 + d0)
                t1 = x1_ + (g1 + d1)
                exn = jnp.where(t0 * t0 + t1 * t1 < R * R, 1.0, 0.0)
                nx0 = (x0_ + g0) + d0
                nx1 = (x1_ + g1) + d1
                aliveN_f = alivef * exn
                newexit_f = alivef - aliveN_f
                nb = newexit_f > 0.0
                ab = aliveN_f > 0.0
                xs0 = jnp.where(nb, x0_, xs0)
                xs1 = jnp.where(nb, x1_, xs1)
                sdf0 = jnp.where(nb, d0, sdf0)
                sdf1 = jnp.where(nb, d1, sdf1)
                epk = jnp.where(nb, ep, epk)
                exitedf = exitedf + newexit_f
                x0_ = jnp.where(ab, nx0, x0_)
                x1_ = jnp.where(ab, nx1, x1_)
                w = (uv * uv + 2.0) * (x0_ * x0_ + x1_ * x1_) - 2.0 * Dim
                A = A + aliveN_f * (w * ep)
                ep = ep * E1
                return (x0_, x1_, aliveN_f, exitedf, A, ep, epk, xs0, xs1, sdf0, sdf1)

            st = (x00, x01, ones, zeros, zeros, ones,
                  zeros, zeros, zeros, zeros, zeros)
            st = lax.fori_loop(1, NSTEP, step, st, unroll=2)
            x0_, x1_, alivef, exitedf, A, ep, epk, xs0, xs1, sdf0, sdf1 = st
            eb = exitedf > 0.0
            xs0 = jnp.where(eb, xs0, x0_)
            xs1 = jnp.where(eb, xs1, x1_)
            epk = jnp.where(eb, epk, ep)
            v_xs0[gl] = xs0
            v_xs1[gl] = xs1
            v_df0[gl] = sdf0
            v_df1[gl] = sdf1
            v_epk[gl] = epk
            v_A[gl] = A
            v_exf[gl] = exitedf
            return carry

        lax.fori_loop(0, SGRP, group, 0, unroll=2)
        pltpu.sync_copy(v_xs0, o_xs0.at[pl.ds(base, SBLK)])
        pltpu.sync_copy(v_xs1, o_xs1.at[pl.ds(base, SBLK)])
        pltpu.sync_copy(v_df0, o_df0.at[pl.ds(base, SBLK)])
        pltpu.sync_copy(v_df1, o_df1.at[pl.ds(base, SBLK)])
        pltpu.sync_copy(v_epk, o_epk.at[pl.ds(base, SBLK)])
        pltpu.sync_copy(v_A, o_A.at[pl.ds(base, SBLK)])
        pltpu.sync_copy(v_exf, o_exf.at[pl.ds(base, SBLK)])


_SC_OUT = [jax.ShapeDtypeStruct((NA,), jnp.float32)] * 7

_sc_phase1 = functools.partial(
    pl.kernel,
    mesh=plsc.VectorSubcoreMesh(core_axis_name="c", subcore_axis_name="s"),
    out_type=_SC_OUT,
    scratch_types=[
        pltpu.VMEM((2, NSTEP, SBLK), jnp.float32),
        pltpu.VMEM((2, SBLK), jnp.float32),
        pltpu.VMEM((16,), jnp.float32),
    ] + [pltpu.VMEM((SBLK,), jnp.float32)] * 7,
)(_sc_phase1_body)


def _rho_tile(xe0, xe1, dr0, dr1, df0, df1):
    a = dr0 * dr0 + dr1 * dr1
    b = 2.0 * (dr0 * df0 + dr1 * df1)
    c = (2.0 * dr0 * xe0 + df0 * df0) + (2.0 * dr1 * xe1 + df1 * df1)
    d = 2.0 * (df0 * xe0 + df1 * xe1)
    e = (xe0 * xe0 + xe1 * xe1) - R ** 2
    p = (8.0 * a * c - 3.0 * (b * b)) / (8.0 * (a * a))
    q = (b * b * b - 4.0 * a * b * c + 8.0 * (a * a) * d) / (8.0 * (a * a * a))
    sign_q = jnp.sign(q)
    D0 = c * c - 3.0 * b * d + 12.0 * a * e
    D1 = (2.0 * (c * c * c) - 9.0 * b * c * d + 27.0 * (b * b) * e
          + 27.0 * a * (d * d) - 72.0 * a * c * e)
    D2 = D1 * D1 - 4.0 * (D0 * D0 * D0)
    signal_D2 = jnp.ceil((jnp.sign(D2) + 1.0) / 2.0)
    QQ = (D1 + jnp.sqrt(jnp.abs(D2))) / 2.0
    Q = jnp.sign(QQ) * jnp.abs(QQ) ** (1.0 / 3.0)
    S_plus = 0.5 * jnp.sqrt(jnp.abs((Q + D0 / Q) / (3.0 * a) - 2.0 * p / 3.0))
    m = jnp.minimum(jnp.sqrt(jnp.abs(D1 * D1 / 4.0 / (D0 * D0 * D0))), 1.0)
    phi = jax.lax.atan2(jnp.sqrt((1.0 - m) * (m + 1.0)), m)
    S_minus = 0.5 * jnp.sqrt(jnp.abs(2.0 * jnp.sqrt(jnp.abs(D0)) * jnp.cos(phi / 3.0) / (3.0 * a)
                                     - 2.0 * p / 3.0))
    S = signal_D2 * S_plus + (1.0 - signal_D2) * S_minus
    temp = -4.0 * (S * S) - 2.0 * p + jnp.abs(q / S)
    sqrt_rho = 0.5 * jnp.sqrt(jnp.abs(temp)) - b / (4.0 * a) - sign_q * S
    bad = (1.0 - sqrt_rho) * sqrt_rho < 0.0
    new_temp = -4.0 * (S * S) - 2.0 * p - jnp.abs(q / S)
    new_sqrt_rho = 0.5 * jnp.sqrt(jnp.abs(new_temp)) - b / (4.0 * a) + sign_q * S
    sqrt_rho_final = jnp.where(bad, new_sqrt_rho, sqrt_rho)
    return sqrt_rho_final * sqrt_rho_final, jnp.abs(sqrt_rho_final)


def _phase2_tail(uf, x00, x01, d00, d01, exit0, w0,
                 xs0, xs1, sdf0, sdf1, epk, A, exited):
    xe0 = jnp.where(exited, xs0, 0.1)
    xe1 = jnp.where(exited, xs1, 0.1)
    dr0 = jnp.where(exited, uf * xs0 * dt, 0.01)
    dr1 = jnp.where(exited, uf * xs1 * dt, 0.01)
    df0 = jnp.where(exited, sdf0, 0.01)
    df1 = jnp.where(exited, sdf1, 0.01)
    rho_q, srho_q = _rho_tile(xe0, xe1, dr0, dr1, df0, df1)
    rho = jnp.where(exited, rho_q, 0.0)
    srho = jnp.where(exited, srho_q, 0.0)
    xk0 = xs0 + uf * xs0 * dt * rho + sdf0 * srho
    xk1 = xs1 + uf * xs1 * dt * rho + sdf1 * srho
    nrm = xk0 * xk0 + xk1 * xk1
    term = epk * (rho * ((uf * uf + 2.0) * nrm - 2.0 * Dim) * dt
                  + jnp.exp(-gamma * dt * rho) * nrm)

    xe0b = jnp.where(exit0, x00, 0.1)
    xe1b = jnp.where(exit0, x01, 0.1)
    dr0b = jnp.where(exit0, uf * x00 * dt, 0.01)
    dr1b = jnp.where(exit0, uf * x01 * dt, 0.01)
    df0b = jnp.where(exit0, d00, 0.01)
    df1b = jnp.where(exit0, d01, 0.01)
    rho0, _ = _rho_tile(xe0b, xe1b, dr0b, dr1b, df0b, df1b)
    coef0 = jnp.where(exit0, rho0, 1.0)
    D0x = jnp.exp(-gamma * dt * coef0)
    return coef0 * w0 * dt + D0x * (dt * A + term)


def _tc_full_body(x0_ref, dwt_ref, u_ref, y_ref):
    uf = u_ref[0, 0]
    x00 = x0_ref[0]
    x01 = x0_ref[1]

    d00 = sigma * dwt_ref[0, 0]
    d01 = sigma * dwt_ref[1, 0]
    g0 = uf * x00 * dt
    g1 = uf * x01 * dt
    t0 = x00 + (g0 + d00)
    t1 = x01 + (g1 + d01)
    exit0 = t0 * t0 + t1 * t1 >= R * R
    w0 = (uf * uf + 2.0) * (x00 * x00 + x01 * x01) - 2.0 * Dim

    zeros = jnp.zeros_like(x00)

    def step(t, st):
        x0_, x1_, alivef, exitedf, A, ep, epk, xs0, xs1, sdf0, sdf1 = st
        alive = alivef > 0.0
        d0 = sigma * dwt_ref[0, t]
        d1 = sigma * dwt_ref[1, t]
        g0 = uf * x0_ * dt
        g1 = uf * x1_ * dt
        t0 = x0_ + (g0 + d0)
        t1 = x1_ + (g1 + d1)
        ex = t0 * t0 + t1 * t1 >= R * R
        nx0 = (x0_ + g0) + d0
        nx1 = (x1_ + g1) + d1
        newexit = alive & ex
        xs0 = jnp.where(newexit, x0_, xs0)
        xs1 = jnp.where(newexit, x1_, xs1)
        sdf0 = jnp.where(newexit, d0, sdf0)
        sdf1 = jnp.where(newexit, d1, sdf1)
        epk = jnp.where(newexit, ep, epk)
        exitedf = jnp.where(newexit, 1.0, exitedf)
        aliveN = alive & jnp.logical_not(ex)
        x0_ = jnp.where(aliveN, nx0, x0_)
        x1_ = jnp.where(aliveN, nx1, x1_)
        w = (uf * uf + 2.0) * (x0_ * x0_ + x1_ * x1_) - 2.0 * Dim
        A = A + jnp.where(aliveN, w * ep, 0.0)
        ep = ep * E1
        alivef = jnp.where(aliveN, 1.0, 0.0)
        return (x0_, x1_, alivef, exitedf, A, ep, epk, xs0, xs1, sdf0, sdf1)

    st = (x00, x01, jnp.ones_like(x00), zeros,
          zeros, jnp.ones_like(x00), zeros, zeros, zeros, zeros, zeros)
    st = jax.lax.fori_loop(1, NSTEP, step, st, unroll=2)
    x0_, x1_, alivef, exitedf, A, ep, epk, xs0, xs1, sdf0, sdf1 = st
    exited = exitedf > 0.0
    xs0 = jnp.where(exited, xs0, x0_)
    xs1 = jnp.where(exited, xs1, x1_)
    epk = jnp.where(exited, epk, ep)

    y_ref[...] = _phase2_tail(uf, x00, x01, d00, d01, exit0, w0,
                              xs0, xs1, sdf0, sdf1, epk, A, exited)


def _tc_phase2_body(x0_ref, dw0_ref, u_ref,
                    xs0_ref, xs1_ref, df0_ref, df1_ref, epk_ref, A_ref, exf_ref,
                    y_ref):
    uf = u_ref[0, 0]
    x00 = x0_ref[0]
    x01 = x0_ref[1]
    d00 = sigma * dw0_ref[0]
    d01 = sigma * dw0_ref[1]
    g0 = uf * x00 * dt
    g1 = uf * x01 * dt
    t0 = x00 + (g0 + d00)
    t1 = x01 + (g1 + d01)
    exit0 = t0 * t0 + t1 * t1 >= R * R
    w0 = (uf * uf + 2.0) * (x00 * x00 + x01 * x01) - 2.0 * Dim

    y_ref[...] = _phase2_tail(uf, x00, x01, d00, d01, exit0, w0,
                              xs0_ref[...], xs1_ref[...], df0_ref[...],
                              df1_ref[...], epk_ref[...], A_ref[...],
                              exf_ref[...] > 0.0)


@jax.jit
def kernel(x0, dw, u):
    rows = NSAMP // LANES
    rows_a = NA // LANES
    blk_a = NA // BLK
    blk_b = NB // BLK

    x0t = x0.transpose(1, 0)
    dwt = dw.transpose(1, 2, 0)
    ub = jnp.full((16,), u, jnp.float32)
    u2d = jnp.reshape(u.astype(jnp.float32), (1, 1))

    x0r = x0t.reshape(2, rows, LANES)
    dwt4 = dwt.reshape(2, NSTEP, rows, LANES)
    dw0r = dwt4[:, 0]

    ph1 = _sc_phase1(x0t, dwt, ub)
    xs0, xs1, df0, df1, epk, A, exf = [v.reshape(rows_a, LANES) for v in ph1]

    pair_b = pl.BlockSpec((2, RB, LANES), lambda i: (0, blk_a + i, 0))
    y_b = pl.pallas_call(
        _tc_full_body,
        grid=(blk_b,),
        in_specs=[pair_b,
                  pl.BlockSpec((2, NSTEP, RB, LANES), lambda i: (0, 0, blk_a + i, 0)),
                  pl.BlockSpec(memory_space=pltpu.SMEM)],
        out_specs=pl.BlockSpec((RB, LANES), lambda i: (i, 0)),
        out_shape=jax.ShapeDtypeStruct((NB // LANES, LANES), jnp.float32),
    )(x0r, dwt4, u2d)

    row_spec = pl.BlockSpec((RB, LANES), lambda i: (i, 0))
    pair_a = pl.BlockSpec((2, RB, LANES), lambda i: (0, i, 0))
    y_a = pl.pallas_call(
        _tc_phase2_body,
        grid=(blk_a,),
        in_specs=[pair_a, pair_a,
                  pl.BlockSpec(memory_space=pltpu.SMEM),
                  row_spec, row_spec, row_spec, row_spec, row_spec,
                  row_spec, row_spec],
        out_specs=row_spec,
        out_shape=jax.ShapeDtypeStruct((rows_a, LANES), jnp.float32),
    )(x0r, dw0r, u2d, xs0, xs1, df0, df1, epk, A, exf)

    y = jnp.concatenate([y_a.reshape(-1), y_b.reshape(-1)])
    return y.reshape(NSAMP, 1)
